# Initial kernel scaffold; baseline (speedup 1.0000x reference)
import jax
import jax.numpy as jnp
from jax.experimental import pallas as pl


def kernel(inputs, embeddings):
    raise NotImplementedError("write your pallas kernel here")


# speedup vs baseline: 11.3322x; 11.3322x over previous
import functools

import jax
import jax.numpy as jnp
from jax import lax
from jax.experimental import pallas as pl
from jax.experimental.pallas import tpu as pltpu
from jax.experimental.pallas import tpu_sc as plsc

_NUM_EMB = 8192
_DIM = 256
_ROWS = 16384
_ROW_TILE = 256
_N_TILES = _ROWS // _ROW_TILE
_COMMIT = 0.25

_SC_NC = 2
_SC_NS = 16
_NW = _SC_NC * _SC_NS
_BPW = _ROWS // _NW
_CH = 128
_NCH = _BPW // _CH


def _dist_argmin_body(x_ref, embt_ref, idx_ref, mind_ref, enorm_ref):
    @pl.when(pl.program_id(0) == 0)
    def _():
        et = embt_ref[...]
        enorm_ref[...] = jnp.sum(et * et, axis=0, keepdims=True)

    x = x_ref[...]
    scores = jnp.dot(x, embt_ref[...], preferred_element_type=jnp.float32)
    dist = enorm_ref[...] - 2.0 * scores
    mind = jnp.min(dist, axis=1)
    idx = jnp.argmin(dist, axis=1).astype(jnp.int32)
    xnorm = jnp.sum(x * x, axis=1)
    idx_ref[...] = idx.reshape(1, 1, _ROW_TILE)
    mind_ref[...] = (mind + xnorm).reshape(1, 1, _ROW_TILE)


def _dist_argmin(flat_x, embt):
    return pl.pallas_call(
        _dist_argmin_body,
        grid=(_N_TILES,),
        in_specs=[
            pl.BlockSpec((_ROW_TILE, _DIM), lambda i: (i, 0)),
            pl.BlockSpec((_DIM, _NUM_EMB), lambda i: (0, 0)),
        ],
        out_specs=[
            pl.BlockSpec((1, 1, _ROW_TILE), lambda i: (i, 0, 0)),
            pl.BlockSpec((1, 1, _ROW_TILE), lambda i: (i, 0, 0)),
        ],
        out_shape=[
            jax.ShapeDtypeStruct((_N_TILES, 1, _ROW_TILE), jnp.int32),
            jax.ShapeDtypeStruct((_N_TILES, 1, _ROW_TILE), jnp.float32),
        ],
        scratch_shapes=[pltpu.VMEM((1, _NUM_EMB), jnp.float32)],
    )(flat_x, embt)


def _sc_gather_body(emb_hbm, idx_hbm, out_hbm, idx_v, rows_a, rows_b,
                    sem_a, sem_b):
    wid = lax.axis_index("s") * _SC_NC + lax.axis_index("c")
    base = wid * _BPW
    pltpu.sync_copy(idx_hbm.at[pl.ds(base, _BPW)], idx_v)
    bufs = (rows_a, rows_b)
    sems = (sem_a, sem_b)

    def issue(c):
        return pltpu.async_copy(
            emb_hbm.at[idx_v.at[pl.ds(c * _CH, _CH)]], bufs[c % 2],
            sems[c % 2])

    cps = [None] * _NCH
    cps[0] = issue(0)
    if _NCH > 1:
        cps[1] = issue(1)
    for c in range(_NCH):
        cps[c].wait()
        pltpu.sync_copy(bufs[c % 2], out_hbm.at[pl.ds(base + c * _CH, _CH)])
        if c + 2 < _NCH:
            cps[c + 2] = issue(c + 2)


def _sc_gather(embeddings, idx_flat):
    gather = pl.kernel(
        _sc_gather_body,
        mesh=plsc.VectorSubcoreMesh(core_axis_name="c", subcore_axis_name="s"),
        out_type=jax.ShapeDtypeStruct((_ROWS, _DIM), jnp.float32),
        scratch_types=[
            pltpu.VMEM((_BPW,), jnp.int32),
            pltpu.VMEM((_CH, _DIM), jnp.float32),
            pltpu.VMEM((_CH, _DIM), jnp.float32),
            pltpu.SemaphoreType.DMA,
            pltpu.SemaphoreType.DMA,
        ],
    )
    return gather(embeddings, idx_flat)


def _stats_body(idx_ref, mind_ref, loss_ref, perp_ref):
    iota = lax.broadcasted_iota(jnp.int32, (1, _NUM_EMB), 1)

    def body(t, counts):
        row = idx_ref[pl.ds(t, 1), :]
        col = row.reshape(_ROW_TILE, 1)
        onehot = (col == iota).astype(jnp.float32)
        return counts + jnp.sum(onehot, axis=0, keepdims=True)

    counts = lax.fori_loop(0, _N_TILES, body,
                           jnp.zeros((1, _NUM_EMB), jnp.float32))
    probs = counts * (1.0 / _ROWS)
    ent = jnp.sum(probs * jnp.log(probs + 1e-10))
    perp_ref[...] = jnp.exp(-ent).reshape(1, 1)
    loss_ref[...] = (jnp.sum(mind_ref[...])
                     * (_COMMIT / (_ROWS * _DIM))).reshape(1, 1)


def _stats(idx, mind):
    return pl.pallas_call(
        _stats_body,
        out_shape=[
            jax.ShapeDtypeStruct((1, 1), jnp.float32),
            jax.ShapeDtypeStruct((1, 1), jnp.float32),
        ],
    )(idx, mind)


def kernel(inputs, embeddings):
    input_shape = inputs.shape
    flat_x = inputs.reshape(_ROWS, _DIM)
    embt = embeddings.T
    idx, mind = _dist_argmin(flat_x, embt)
    idx = idx.reshape(_N_TILES, _ROW_TILE)
    mind = mind.reshape(_N_TILES, _ROW_TILE)
    quantized = _sc_gather(embeddings, idx.reshape(_ROWS))
    loss, perp = _stats(idx, mind)
    quantized_st = quantized.reshape(input_shape)
    return (quantized_st, loss.reshape(()), perp.reshape(()), idx)
